# baseline (speedup 1.0000x reference)
import jax
import jax.numpy as jnp
from jax.experimental import pallas as pl


_LANES = 512
_BLOCK_ROWS = 2000


def _broadcast_body(w_ref, o_ref):
    o_ref[...] = jnp.broadcast_to(w_ref[...], o_ref.shape)


def kernel(edge_index, W):
    E = edge_index.shape[1]
    D = W.shape[1]
    rows = E * D // _LANES
    w_wide = jnp.reshape(jnp.broadcast_to(W[0], (_LANES // D, D)), (1, _LANES))
    out = pl.pallas_call(
        _broadcast_body,
        grid=(rows // _BLOCK_ROWS,),
        in_specs=[pl.BlockSpec((1, _LANES), lambda i: (0, 0))],
        out_specs=pl.BlockSpec((_BLOCK_ROWS, _LANES), lambda i: (i, 0)),
        out_shape=jax.ShapeDtypeStruct((rows, _LANES), jnp.float32),
    )(w_wide)
    return out.reshape(E, D)


# speedup vs baseline: 1.3902x; 1.3902x over previous
import jax
import jax.numpy as jnp
from jax.experimental import pallas as pl
from jax.experimental.pallas import tpu as pltpu


_R = 8000
_WINDOW = 16


def _body(w_ref, o_ref, buf, sem):
    buf[...] = jnp.broadcast_to(w_ref[...], buf.shape)
    n = o_ref.shape[0] // _R
    for k in range(n):
        pltpu.make_async_copy(buf, o_ref.at[pl.ds(k * _R, _R)], sem).start()
        if k >= _WINDOW:
            pltpu.make_async_copy(
                buf, o_ref.at[pl.ds((k - _WINDOW) * _R, _R)], sem).wait()
    for k in range(max(n - _WINDOW, 0), n):
        pltpu.make_async_copy(buf, o_ref.at[pl.ds(k * _R, _R)], sem).wait()


def kernel(edge_index, W):
    E = edge_index.shape[1]
    D = W.shape[1]
    return pl.pallas_call(
        _body,
        in_specs=[pl.BlockSpec(memory_space=pltpu.MemorySpace.VMEM)],
        out_specs=pl.BlockSpec(memory_space=pltpu.MemorySpace.HBM),
        out_shape=jax.ShapeDtypeStruct((E, D), jnp.float32),
        scratch_shapes=[
            pltpu.MemorySpace.VMEM((_R, D), jnp.float32),
            pltpu.SemaphoreType.DMA,
        ],
    )(W)
